# baseline (speedup 1.0000x reference)
import functools

import numpy as np

import jax
import jax.numpy as jnp
from jax.experimental import pallas as pl
from jax.experimental.pallas import tpu as pltpu

_KS0 = 0
_KS1 = 42
_KS2 = _KS0 ^ _KS1 ^ 0x1BD11BDA

_ROT = ((13, 15, 26, 6), (17, 29, 16, 24))

_CHUNK = 1024
_S_UNROLL = 4


def _u32(x):
    return jnp.asarray(x, dtype=jnp.uint32)


def _threefry2x32(x0, x1):
    ks = (_u32(_KS0), _u32(_KS1), _u32(_KS2))
    x1 = x1 + ks[1]
    for i in range(5):
        rots = _ROT[i % 2]
        for r in rots:
            x0 = x0 + x1
            x1 = (x1 << _u32(r)) | (x1 >> _u32(32 - r))
            x1 = x0 ^ x1
        x0 = x0 + ks[(i + 1) % 3]
        x1 = x1 + ks[(i + 2) % 3] + _u32(i + 1)
    return x0, x1


def _matmul_kernel(x_ref, w_ref, b_ref, c_ref, z_ref, *, kblk):
    i = pl.program_id(0)
    logits = (
        jnp.dot(x_ref[...], w_ref[...], preferred_element_type=jnp.float32)
        + b_ref[...]
    )
    el = jnp.exp(logits)
    c_ref[...] = jnp.exp(-logits)
    acc = el[:, 0:128]
    for j in range(1, kblk // 128):
        acc = acc + el[:, j * 128:(j + 1) * 128]

    @pl.when(i == 0)
    def _():
        z_ref[...] = acc

    @pl.when(i != 0)
    def _():
        z_ref[...] = z_ref[...] + acc


def _sampler_kernel(c_ref, blo_ref, bhi_ref, act_ref, cwin_ref, *, m, nch):
    kiota = (
        jax.lax.broadcasted_iota(jnp.int32, (8, 128), 0) * 128
        + jax.lax.broadcasted_iota(jnp.int32, (8, 128), 1)
    )
    kiota_u = kiota.astype(jnp.uint32)
    inf = jnp.float32(np.inf)

    def s_body(sg, _):
        s0 = sg * _S_UNROLL

        def init(j):
            del j
            return (
                jnp.full((8, 128), inf, jnp.float32),
                jnp.zeros((8, 128), jnp.int32),
                jnp.ones((8, 128), jnp.float32),
            )

        los = [_u32(blo_ref[0, 0, s0 + j]) for j in range(_S_UNROLL)]
        his = [_u32(bhi_ref[0, 0, s0 + j]) for j in range(_S_UNROLL)]

        def ch_body(ch, state):
            c = c_ref[0, ch]
            kv_u = kiota_u + _u32(ch * _CHUNK)
            kv_i = kiota + ch * _CHUNK
            out = []
            for j in range(_S_UNROLL):
                bw, bi, bc = state[j]
                lo = los[j] + kv_u
                carry = (lo < los[j]).astype(jnp.uint32)
                hi = his[j] + carry
                o0, o1 = _threefry2x32(hi, lo)
                bits = o0 ^ o1
                n = _u32(1 << 23) - (bits >> _u32(9))
                w = n.astype(jnp.int32).astype(jnp.float32) * c
                pred = w < bw
                bw = jnp.where(pred, w, bw)
                bi = jnp.where(pred, kv_i, bi)
                bc = jnp.where(pred, c, bc)
                out.append((bw, bi, bc))
            return tuple(out)

        state = jax.lax.fori_loop(
            0, nch, ch_body, tuple(init(j) for j in range(_S_UNROLL))
        )

        for j in range(_S_UNROLL):
            bw, bi, bc = state[j]
            wmin = jnp.min(bw)
            elig = bw == wmin
            idxm = jnp.min(jnp.where(elig, bi, jnp.int32(0x7FFFFFFF)))
            cm = jnp.max(jnp.where(elig & (bi == idxm), bc, jnp.float32(0.0)))
            act_ref[0, 0, s0 + j] = idxm
            cwin_ref[0, 0, s0 + j] = cm
        return 0

    jax.lax.fori_loop(0, m // _S_UNROLL, s_body, 0)


@jax.jit
def kernel(x, W, b):
    B, D = x.shape
    K = W.shape[1]
    m = B
    kp = pl.cdiv(K, _CHUNK) * _CHUNK
    nch = kp // _CHUNK
    kblk = 2048 if kp % 2048 == 0 else _CHUNK
    nkb = kp // kblk

    Wp = jnp.pad(W, ((0, 0), (0, kp - K)))
    bp = jnp.pad(b, (0, kp - K), constant_values=-80.0).reshape(1, kp)

    C, z128 = pl.pallas_call(
        functools.partial(_matmul_kernel, kblk=kblk),
        grid=(nkb,),
        in_specs=[
            pl.BlockSpec((B, D), lambda i: (0, 0)),
            pl.BlockSpec((D, kblk), lambda i: (0, i)),
            pl.BlockSpec((1, kblk), lambda i: (0, i)),
        ],
        out_specs=[
            pl.BlockSpec((B, kblk), lambda i: (0, i)),
            pl.BlockSpec((B, 128), lambda i: (0, 0)),
        ],
        out_shape=[
            jax.ShapeDtypeStruct((B, kp), jnp.float32),
            jax.ShapeDtypeStruct((B, 128), jnp.float32),
        ],
        compiler_params=pltpu.CompilerParams(
            dimension_semantics=("arbitrary",),
        ),
    )(x, Wp, bp)

    base = (
        np.arange(B, dtype=np.uint64)[:, None] * np.uint64(K)
        + np.arange(m, dtype=np.uint64)[None, :] * np.uint64(B * K)
    )
    base_lo = jnp.asarray(
        (base & np.uint64(0xFFFFFFFF)).astype(np.uint32).view(np.int32)
    ).reshape(B, 1, m)
    base_hi = jnp.asarray(
        (base >> np.uint64(32)).astype(np.uint32).view(np.int32)
    ).reshape(B, 1, m)

    C4 = C.reshape(B, nch, 8, 128)

    act_t, cwin_t = pl.pallas_call(
        functools.partial(_sampler_kernel, m=m, nch=nch),
        grid=(B,),
        in_specs=[
            pl.BlockSpec((1, nch, 8, 128), lambda i: (i, 0, 0, 0)),
            pl.BlockSpec((1, 1, m), lambda i: (i, 0, 0), memory_space=pltpu.SMEM),
            pl.BlockSpec((1, 1, m), lambda i: (i, 0, 0), memory_space=pltpu.SMEM),
        ],
        out_specs=[
            pl.BlockSpec((1, 1, m), lambda i: (i, 0, 0), memory_space=pltpu.SMEM),
            pl.BlockSpec((1, 1, m), lambda i: (i, 0, 0), memory_space=pltpu.SMEM),
        ],
        out_shape=[
            jax.ShapeDtypeStruct((B, 1, m), jnp.int32),
            jax.ShapeDtypeStruct((B, 1, m), jnp.float32),
        ],
        compiler_params=pltpu.CompilerParams(
            dimension_semantics=("parallel",),
        ),
    )(C4, base_lo, base_hi)

    act_t = act_t.reshape(B, m)
    cwin_t = cwin_t.reshape(B, m)
    Z = jnp.sum(z128, axis=1)
    p_t = 1.0 / (cwin_t * Z[:, None])
    return act_t.T, p_t.T
